# baseline (speedup 1.0000x reference)
import functools

import jax
import jax.numpy as jnp
from jax import lax
from jax.experimental import pallas as pl
from jax.experimental.pallas import tpu as pltpu
from jax.experimental.pallas import tpu_sc as plsc

_E, _N, _D, _R = 320000, 10000, 128, 16
_B = 80
_NROWS = _E // _B
_NW = 32
_WBASE = _NROWS // _NW
_WREM = _NROWS % _NW
_NPT = _N // 16
_ZR = 25


def _swish(v):
    return v * jax.nn.sigmoid(v)


_EB = 16000


def _edge_body(rbf_ref, x_ref, w_ref, o_ref):
    o_ref[...] = (
        jnp.dot(rbf_ref[...], w_ref[...], preferred_element_type=jnp.float32)
        * x_ref[...]
    )


def _edge_embed(rbf, x, W_rbf):
    return pl.pallas_call(
        _edge_body,
        grid=(_E // _EB,),
        in_specs=[
            pl.BlockSpec((_EB, _R), lambda i: (i, 0)),
            pl.BlockSpec((_EB, _D), lambda i: (i, 0)),
            pl.BlockSpec((_R, _D), lambda i: (0, 0)),
        ],
        out_specs=pl.BlockSpec((_EB, _D), lambda i: (i, 0)),
        out_shape=jax.ShapeDtypeStruct((_E, _D), jnp.float32),
    )(rbf, x, W_rbf)


_NBUF = 4


@functools.cache
def _make_scatter_kernel():
    mesh = plsc.VectorSubcoreMesh(core_axis_name="c", subcore_axis_name="s")
    return functools.partial(
        pl.kernel,
        out_type=jax.ShapeDtypeStruct((2, _N, _D), jnp.float32),
        mesh=mesh,
        compiler_params=pltpu.CompilerParams(use_tc_tiling_on_sc=False),
        scratch_types=[
            pltpu.VMEM((_NBUF, _B), jnp.int32),
            pltpu.VMEM((_NBUF, _B, _D), jnp.float32),
            pltpu.VMEM((_ZR, _D), jnp.float32),
            pltpu.VMEM_SHARED((_N, _D), jnp.float32),
            pltpu.SemaphoreType.DMA((_NBUF,)),
        ],
    )(_scatter_body)


def _scatter_body(h_hbm, idx_hbm, out_hbm, idx_v, h_v, z_v, acc, sem):
    c = lax.axis_index("c")
    s = lax.axis_index("s")
    w = c * 16 + s

    def _zb(i, carry):
        z_v[i // 8, pl.ds((i % 8) * 16, 16)] = jnp.zeros((16,), jnp.float32)
        return carry

    lax.fori_loop(0, _ZR * 8, _zb, 0)
    base = s * _NPT
    for j in range(_NPT // _ZR):
        pltpu.sync_copy(z_v, acc.at[pl.ds(base + j * _ZR, _ZR)])
    plsc.subcore_barrier()

    start = w * _WBASE + jnp.minimum(w, _WREM)
    cnt = _WBASE + jnp.where(w < _WREM, 1, 0)

    def _fire(i, b):
        row = start + i
        pltpu.async_copy(idx_hbm.at[row], idx_v.at[b], sem.at[b])
        pltpu.async_copy(h_hbm.at[row], h_v.at[b], sem.at[b])

    def _drain(i, b):
        row = start + i
        pltpu.make_async_copy(idx_hbm.at[row], idx_v.at[b], sem.at[b]).wait()
        pltpu.make_async_copy(h_hbm.at[row], h_v.at[b], sem.at[b]).wait()

    for b in range(_NBUF):
        @pl.when(b < cnt)
        def _():
            _fire(b, b)

    _outer = (_WBASE + _NBUF) // _NBUF

    def _body(j, carry):
        for b in range(_NBUF):
            i = j * _NBUF + b

            @pl.when(i < cnt)
            def _():
                _drain(i, b)
                pltpu.sync_copy(h_v.at[b], acc.at[idx_v.at[b]], add=True)

                @pl.when(i + _NBUF < cnt)
                def _():
                    _fire(i + _NBUF, b)
        return carry

    lax.fori_loop(0, _outer, _body, 0)
    plsc.subcore_barrier()

    pltpu.sync_copy(acc.at[pl.ds(base, _NPT)], out_hbm.at[c, pl.ds(base, _NPT)])


_NB = 1000


def _mlp_body(p_ref, w1_ref, b1_ref, w2_ref, b2_ref, w3_ref, b3_ref, wo_ref,
              o_ref):
    agg = p_ref[0] + p_ref[1]
    h = _swish(jnp.dot(agg, w1_ref[...], preferred_element_type=jnp.float32)
               + b1_ref[...])
    h = _swish(jnp.dot(h, w2_ref[...], preferred_element_type=jnp.float32)
               + b2_ref[...])
    h = _swish(jnp.dot(h, w3_ref[...], preferred_element_type=jnp.float32)
               + b3_ref[...])
    o_ref[...] = jnp.dot(h, wo_ref[...], preferred_element_type=jnp.float32)


def _mlp(partials, W1, b1, W2, b2, W3, b3, W_out):
    O = W_out.shape[1]
    return pl.pallas_call(
        _mlp_body,
        grid=(_N // _NB,),
        in_specs=[
            pl.BlockSpec((2, _NB, _D), lambda i: (0, i, 0)),
            pl.BlockSpec((_D, _D), lambda i: (0, 0)),
            pl.BlockSpec((1, _D), lambda i: (0, 0)),
            pl.BlockSpec((_D, _D), lambda i: (0, 0)),
            pl.BlockSpec((1, _D), lambda i: (0, 0)),
            pl.BlockSpec((_D, _D), lambda i: (0, 0)),
            pl.BlockSpec((1, _D), lambda i: (0, 0)),
            pl.BlockSpec((_D, O), lambda i: (0, 0)),
        ],
        out_specs=pl.BlockSpec((_NB, O), lambda i: (i, 0)),
        out_shape=jax.ShapeDtypeStruct((_N, O), jnp.float32),
    )(partials, W1, b1.reshape(1, _D), W2, b2.reshape(1, _D), W3,
      b3.reshape(1, _D), W_out)


def kernel(x, rbf, idx_i, num_nodes, W_rbf, W1, b1, W2, b2, W3, b3, W_out):
    h = _edge_embed(rbf, x, W_rbf)
    idx = (idx_i.astype(jnp.int32) % jnp.int32(num_nodes)).reshape(_NROWS, _B)
    partials = _make_scatter_kernel()(h.reshape(_NROWS, _B, _D), idx)
    return _mlp(partials, W1, b1, W2, b2, W3, b3, W_out)


# speedup vs baseline: 1.0063x; 1.0063x over previous
import functools

import jax
import jax.numpy as jnp
from jax import lax
from jax.experimental import pallas as pl
from jax.experimental.pallas import tpu as pltpu
from jax.experimental.pallas import tpu_sc as plsc

_E, _N, _D, _R = 320000, 10000, 128, 16
_B = 80
_NROWS = _E // _B
_NW = 32
_NPT = _N // 16
_ZR = 25
_NBUF = 4
_NC = 2
_CROWS = _NROWS // _NC
_CE = _E // _NC


def _swish(v):
    return v * jax.nn.sigmoid(v)


_EB = 8000


def _edge_body(rbf_ref, x_ref, w_ref, o_ref):
    o_ref[...] = (
        jnp.dot(rbf_ref[...], w_ref[...], preferred_element_type=jnp.float32)
        * x_ref[...]
    )


def _edge_embed(rbf, x, W_rbf, chunk):
    nblk = _CE // _EB
    off = chunk * nblk
    return pl.pallas_call(
        _edge_body,
        grid=(nblk,),
        in_specs=[
            pl.BlockSpec((_EB, _R), lambda i: (off + i, 0)),
            pl.BlockSpec((_EB, _D), lambda i: (off + i, 0)),
            pl.BlockSpec((_R, _D), lambda i: (0, 0)),
        ],
        out_specs=pl.BlockSpec((_EB, _D), lambda i: (i, 0)),
        out_shape=jax.ShapeDtypeStruct((_CE, _D), jnp.float32),
    )(rbf, x, W_rbf)


@functools.cache
def _make_scatter_kernel(row0):
    mesh = plsc.VectorSubcoreMesh(core_axis_name="c", subcore_axis_name="s")
    wbase, wrem = _CROWS // _NW, _CROWS % _NW
    maxb = wbase + (1 if wrem else 0)
    outer = (maxb + _NBUF - 1) // _NBUF

    def _scatter_body(h_hbm, idx_hbm, out_hbm, idx_v, h_v, z_v, acc, sem):
        c = lax.axis_index("c")
        s = lax.axis_index("s")
        w = c * 16 + s

        def _zb(i, carry):
            z_v[i // 8, pl.ds((i % 8) * 16, 16)] = jnp.zeros((16,), jnp.float32)
            return carry

        lax.fori_loop(0, _ZR * 8, _zb, 0)
        base = s * _NPT
        for j in range(_NPT // _ZR):
            pltpu.sync_copy(z_v, acc.at[pl.ds(base + j * _ZR, _ZR)])
        plsc.subcore_barrier()

        start = w * wbase + jnp.minimum(w, wrem)
        cnt = wbase + jnp.where(w < wrem, 1, 0)

        def _fire(i, b):
            row = start + i
            pltpu.async_copy(idx_hbm.at[row0 + row], idx_v.at[b], sem.at[b])
            pltpu.async_copy(h_hbm.at[row], h_v.at[b], sem.at[b])

        def _drain(i, b):
            row = start + i
            pltpu.make_async_copy(
                idx_hbm.at[row0 + row], idx_v.at[b], sem.at[b]).wait()
            pltpu.make_async_copy(h_hbm.at[row], h_v.at[b], sem.at[b]).wait()

        for b in range(_NBUF):
            @pl.when(b < cnt)
            def _():
                _fire(b, b)

        def _body(j, carry):
            for b in range(_NBUF):
                i = j * _NBUF + b

                @pl.when(i < cnt)
                def _():
                    _drain(i, b)
                    pltpu.sync_copy(h_v.at[b], acc.at[idx_v.at[b]], add=True)

                    @pl.when(i + _NBUF < cnt)
                    def _():
                        _fire(i + _NBUF, b)
            return carry

        lax.fori_loop(0, outer, _body, 0)
        plsc.subcore_barrier()

        pltpu.sync_copy(acc.at[pl.ds(base, _NPT)],
                        out_hbm.at[c, pl.ds(base, _NPT)])

    return functools.partial(
        pl.kernel,
        out_type=jax.ShapeDtypeStruct((2, _N, _D), jnp.float32),
        mesh=mesh,
        compiler_params=pltpu.CompilerParams(use_tc_tiling_on_sc=False),
        scratch_types=[
            pltpu.VMEM((_NBUF, _B), jnp.int32),
            pltpu.VMEM((_NBUF, _B, _D), jnp.float32),
            pltpu.VMEM((_ZR, _D), jnp.float32),
            pltpu.VMEM_SHARED((_N, _D), jnp.float32),
            pltpu.SemaphoreType.DMA((_NBUF,)),
        ],
    )(_scatter_body)


_NB = 1000


def _mlp_body(*refs):
    p_refs = refs[:_NC]
    w1_ref, b1_ref, w2_ref, b2_ref, w3_ref, b3_ref, wo_ref, o_ref = refs[_NC:]
    agg = p_refs[0][0] + p_refs[0][1]
    for p in p_refs[1:]:
        agg = agg + p[0] + p[1]
    h = _swish(jnp.dot(agg, w1_ref[...], preferred_element_type=jnp.float32)
               + b1_ref[...])
    h = _swish(jnp.dot(h, w2_ref[...], preferred_element_type=jnp.float32)
               + b2_ref[...])
    h = _swish(jnp.dot(h, w3_ref[...], preferred_element_type=jnp.float32)
               + b3_ref[...])
    o_ref[...] = jnp.dot(h, wo_ref[...], preferred_element_type=jnp.float32)


def _mlp(partial_list, W1, b1, W2, b2, W3, b3, W_out):
    O = W_out.shape[1]
    return pl.pallas_call(
        _mlp_body,
        grid=(_N // _NB,),
        in_specs=(
            [pl.BlockSpec((2, _NB, _D), lambda i: (0, i, 0))] * _NC
            + [
                pl.BlockSpec((_D, _D), lambda i: (0, 0)),
                pl.BlockSpec((1, _D), lambda i: (0, 0)),
                pl.BlockSpec((_D, _D), lambda i: (0, 0)),
                pl.BlockSpec((1, _D), lambda i: (0, 0)),
                pl.BlockSpec((_D, _D), lambda i: (0, 0)),
                pl.BlockSpec((1, _D), lambda i: (0, 0)),
                pl.BlockSpec((_D, O), lambda i: (0, 0)),
            ]
        ),
        out_specs=pl.BlockSpec((_NB, O), lambda i: (i, 0)),
        out_shape=jax.ShapeDtypeStruct((_N, O), jnp.float32),
    )(*partial_list, W1, b1.reshape(1, _D), W2, b2.reshape(1, _D), W3,
      b3.reshape(1, _D), W_out)


def kernel(x, rbf, idx_i, num_nodes, W_rbf, W1, b1, W2, b2, W3, b3, W_out):
    idx = (idx_i.astype(jnp.int32) % jnp.int32(num_nodes)).reshape(_NROWS, _B)
    partial_list = []
    for k in range(_NC):
        h_k = _edge_embed(rbf, x, W_rbf, k)
        partial_list.append(
            _make_scatter_kernel(k * _CROWS)(h_k.reshape(_CROWS, _B, _D), idx))
    return _mlp(partial_list, W1, b1, W2, b2, W3, b3, W_out)


# speedup vs baseline: 1.0097x; 1.0033x over previous
import functools

import jax
import jax.numpy as jnp
from jax import lax
from jax.experimental import pallas as pl
from jax.experimental.pallas import tpu as pltpu
from jax.experimental.pallas import tpu_sc as plsc

_E, _N, _D, _R = 320000, 10000, 128, 16
_B = 80
_NROWS = _E // _B
_NW = 32
_NPT = _N // 16
_ZR = 25
_NBUF = 4
_NC = 2
_CROWS = _NROWS // _NC
_CE = _E // _NC


def _swish(v):
    return v * jax.nn.sigmoid(v)


_EB = 8000


def _edge_body(rbf_ref, x_ref, w_ref, o_ref):
    o_ref[...] = (
        jnp.dot(rbf_ref[...], w_ref[...], preferred_element_type=jnp.float32)
        * x_ref[...]
    )


def _edge_embed(rbf, x, W_rbf, chunk):
    nblk = _CE // _EB
    off = chunk * nblk
    return pl.pallas_call(
        _edge_body,
        grid=(nblk,),
        in_specs=[
            pl.BlockSpec((_EB, _R), lambda i: (off + i, 0)),
            pl.BlockSpec((_EB, _D), lambda i: (off + i, 0)),
            pl.BlockSpec((_R, _D), lambda i: (0, 0)),
        ],
        out_specs=pl.BlockSpec((_EB, _D), lambda i: (i, 0)),
        out_shape=jax.ShapeDtypeStruct((_CE, _D), jnp.float32),
    )(rbf, x, W_rbf)


@functools.cache
def _make_scatter_kernel(row0):
    mesh = plsc.VectorSubcoreMesh(core_axis_name="c", subcore_axis_name="s")
    wbase, wrem = _CROWS // _NW, _CROWS % _NW
    maxb = wbase + (1 if wrem else 0)
    outer = (maxb + _NBUF - 1) // _NBUF

    def _scatter_body(h_hbm, idx_hbm, out_hbm, idx_v, h_v, z_v, acc, sem):
        c = lax.axis_index("c")
        s = lax.axis_index("s")
        w = c * 16 + s

        def _zb(i, carry):
            z_v[i // 8, pl.ds((i % 8) * 16, 16)] = jnp.zeros((16,), jnp.float32)
            return carry

        lax.fori_loop(0, _ZR * 8, _zb, 0)
        base = s * _NPT
        for j in range(_NPT // _ZR):
            pltpu.async_copy(z_v, acc.at[pl.ds(base + j * _ZR, _ZR)],
                             sem.at[0])
        for j in range(_NPT // _ZR):
            pltpu.make_async_copy(z_v, acc.at[pl.ds(base + j * _ZR, _ZR)],
                                  sem.at[0]).wait()
        plsc.subcore_barrier()

        start = w * wbase + jnp.minimum(w, wrem)
        cnt = wbase + jnp.where(w < wrem, 1, 0)

        def _fire(i, b):
            row = start + i
            pltpu.async_copy(idx_hbm.at[row0 + row], idx_v.at[b], sem.at[b])
            pltpu.async_copy(h_hbm.at[row], h_v.at[b], sem.at[b])

        def _drain(i, b):
            row = start + i
            pltpu.make_async_copy(
                idx_hbm.at[row0 + row], idx_v.at[b], sem.at[b]).wait()
            pltpu.make_async_copy(h_hbm.at[row], h_v.at[b], sem.at[b]).wait()

        for b in range(_NBUF):
            @pl.when(b < cnt)
            def _():
                _fire(b, b)

        def _body(j, carry):
            for b in range(_NBUF):
                i = j * _NBUF + b

                @pl.when(i < cnt)
                def _():
                    _drain(i, b)
                    pltpu.sync_copy(h_v.at[b], acc.at[idx_v.at[b]], add=True)

                    @pl.when(i + _NBUF < cnt)
                    def _():
                        _fire(i + _NBUF, b)
            return carry

        lax.fori_loop(0, outer, _body, 0)
        plsc.subcore_barrier()

        pltpu.sync_copy(acc.at[pl.ds(base, _NPT)],
                        out_hbm.at[c, pl.ds(base, _NPT)])

    return functools.partial(
        pl.kernel,
        out_type=jax.ShapeDtypeStruct((2, _N, _D), jnp.float32),
        mesh=mesh,
        compiler_params=pltpu.CompilerParams(use_tc_tiling_on_sc=False),
        scratch_types=[
            pltpu.VMEM((_NBUF, _B), jnp.int32),
            pltpu.VMEM((_NBUF, _B, _D), jnp.float32),
            pltpu.VMEM((_ZR, _D), jnp.float32),
            pltpu.VMEM_SHARED((_N, _D), jnp.float32),
            pltpu.SemaphoreType.DMA((_NBUF,)),
        ],
    )(_scatter_body)


_NB = 1000


def _mlp_body(*refs):
    p_refs = refs[:_NC]
    w1_ref, b1_ref, w2_ref, b2_ref, w3_ref, b3_ref, wo_ref, o_ref = refs[_NC:]
    agg = p_refs[0][0] + p_refs[0][1]
    for p in p_refs[1:]:
        agg = agg + p[0] + p[1]
    h = _swish(jnp.dot(agg, w1_ref[...], preferred_element_type=jnp.float32)
               + b1_ref[...])
    h = _swish(jnp.dot(h, w2_ref[...], preferred_element_type=jnp.float32)
               + b2_ref[...])
    h = _swish(jnp.dot(h, w3_ref[...], preferred_element_type=jnp.float32)
               + b3_ref[...])
    o_ref[...] = jnp.dot(h, wo_ref[...], preferred_element_type=jnp.float32)


def _mlp(partial_list, W1, b1, W2, b2, W3, b3, W_out):
    O = W_out.shape[1]
    return pl.pallas_call(
        _mlp_body,
        grid=(_N // _NB,),
        in_specs=(
            [pl.BlockSpec((2, _NB, _D), lambda i: (0, i, 0))] * _NC
            + [
                pl.BlockSpec((_D, _D), lambda i: (0, 0)),
                pl.BlockSpec((1, _D), lambda i: (0, 0)),
                pl.BlockSpec((_D, _D), lambda i: (0, 0)),
                pl.BlockSpec((1, _D), lambda i: (0, 0)),
                pl.BlockSpec((_D, _D), lambda i: (0, 0)),
                pl.BlockSpec((1, _D), lambda i: (0, 0)),
                pl.BlockSpec((_D, O), lambda i: (0, 0)),
            ]
        ),
        out_specs=pl.BlockSpec((_NB, O), lambda i: (i, 0)),
        out_shape=jax.ShapeDtypeStruct((_N, O), jnp.float32),
    )(*partial_list, W1, b1.reshape(1, _D), W2, b2.reshape(1, _D), W3,
      b3.reshape(1, _D), W_out)


def kernel(x, rbf, idx_i, num_nodes, W_rbf, W1, b1, W2, b2, W3, b3, W_out):
    idx = (idx_i.astype(jnp.int32) % jnp.int32(num_nodes)).reshape(_NROWS, _B)
    partial_list = []
    for k in range(_NC):
        h_k = _edge_embed(rbf, x, W_rbf, k)
        partial_list.append(
            _make_scatter_kernel(k * _CROWS)(h_k.reshape(_CROWS, _B, _D), idx))
    return _mlp(partial_list, W1, b1, W2, b2, W3, b3, W_out)


# speedup vs baseline: 1.5829x; 1.5678x over previous
import functools

import jax
import jax.numpy as jnp
from jax import lax
from jax.experimental import pallas as pl
from jax.experimental.pallas import tpu as pltpu
from jax.experimental.pallas import tpu_sc as plsc

_E, _N, _D, _R = 320000, 10000, 128, 16
_B = 80
_NROWS = _E // _B
_NW = 32
_NPT = _N // 16
_ZR = 25
_NBUF = 4
_NC = 2
_CROWS = _NROWS // _NC
_CE = _E // _NC


def _swish(v):
    return v * jax.nn.sigmoid(v)


_EB = 16000


def _edge_body(rbft_ref, x_ref, w_ref, o_ref):
    o_ref[...] = (
        lax.dot_general(rbft_ref[...], w_ref[...],
                        dimension_numbers=(((0,), (0,)), ((), ())),
                        preferred_element_type=jnp.float32)
        * x_ref[...]
    )


def _edge_embed(rbf_t, x, W_rbf, chunk):
    nblk = _CE // _EB
    off = chunk * nblk
    return pl.pallas_call(
        _edge_body,
        grid=(nblk,),
        in_specs=[
            pl.BlockSpec((_R, _EB), lambda i: (0, off + i)),
            pl.BlockSpec((_EB, _D), lambda i: (off + i, 0)),
            pl.BlockSpec((_R, _D), lambda i: (0, 0)),
        ],
        out_specs=pl.BlockSpec((_EB, _D), lambda i: (i, 0)),
        out_shape=jax.ShapeDtypeStruct((_CE, _D), jnp.float32),
    )(rbf_t, x, W_rbf)


@functools.cache
def _make_scatter_kernel(row0):
    mesh = plsc.VectorSubcoreMesh(core_axis_name="c", subcore_axis_name="s")
    wbase, wrem = _CROWS // _NW, _CROWS % _NW
    maxb = wbase + (1 if wrem else 0)
    outer = (maxb + _NBUF - 1) // _NBUF

    def _scatter_body(h_hbm, idx_hbm, out_hbm, idx_v, h_v, z_v, acc, sem):
        c = lax.axis_index("c")
        s = lax.axis_index("s")
        w = c * 16 + s

        def _zb(i, carry):
            z_v[i // 8, pl.ds((i % 8) * 16, 16)] = jnp.zeros((16,), jnp.float32)
            return carry

        lax.fori_loop(0, _ZR * 8, _zb, 0)
        base = s * _NPT
        for j in range(_NPT // _ZR):
            pltpu.async_copy(z_v, acc.at[pl.ds(base + j * _ZR, _ZR)],
                             sem.at[0])
        for j in range(_NPT // _ZR):
            pltpu.make_async_copy(z_v, acc.at[pl.ds(base + j * _ZR, _ZR)],
                                  sem.at[0]).wait()
        plsc.subcore_barrier()

        start = w * wbase + jnp.minimum(w, wrem)
        cnt = wbase + jnp.where(w < wrem, 1, 0)

        def _fire(i, b):
            row = start + i
            pltpu.async_copy(idx_hbm.at[row0 + row], idx_v.at[b], sem.at[b])
            pltpu.async_copy(h_hbm.at[row], h_v.at[b], sem.at[b])

        def _drain(i, b):
            row = start + i
            pltpu.make_async_copy(
                idx_hbm.at[row0 + row], idx_v.at[b], sem.at[b]).wait()
            pltpu.make_async_copy(h_hbm.at[row], h_v.at[b], sem.at[b]).wait()

        for b in range(_NBUF):
            @pl.when(b < cnt)
            def _():
                _fire(b, b)

        def _body(j, carry):
            for b in range(_NBUF):
                i = j * _NBUF + b

                @pl.when(i < cnt)
                def _():
                    _drain(i, b)
                    pltpu.sync_copy(h_v.at[b], acc.at[idx_v.at[b]], add=True)

                    @pl.when(i + _NBUF < cnt)
                    def _():
                        _fire(i + _NBUF, b)
            return carry

        lax.fori_loop(0, outer, _body, 0)
        plsc.subcore_barrier()

        pltpu.sync_copy(acc.at[pl.ds(base, _NPT)],
                        out_hbm.at[c, pl.ds(base, _NPT)])

    return functools.partial(
        pl.kernel,
        out_type=jax.ShapeDtypeStruct((2, _N, _D), jnp.float32),
        mesh=mesh,
        compiler_params=pltpu.CompilerParams(use_tc_tiling_on_sc=False),
        scratch_types=[
            pltpu.VMEM((_NBUF, _B), jnp.int32),
            pltpu.VMEM((_NBUF, _B, _D), jnp.float32),
            pltpu.VMEM((_ZR, _D), jnp.float32),
            pltpu.VMEM_SHARED((_N, _D), jnp.float32),
            pltpu.SemaphoreType.DMA((_NBUF,)),
        ],
    )(_scatter_body)


_NB = 1000


def _mlp_body(*refs):
    p_refs = refs[:_NC]
    w1_ref, b1_ref, w2_ref, b2_ref, w3_ref, b3_ref, wo_ref, o_ref = refs[_NC:]
    agg = p_refs[0][0] + p_refs[0][1]
    for p in p_refs[1:]:
        agg = agg + p[0] + p[1]
    h = _swish(jnp.dot(agg, w1_ref[...], preferred_element_type=jnp.float32)
               + b1_ref[...])
    h = _swish(jnp.dot(h, w2_ref[...], preferred_element_type=jnp.float32)
               + b2_ref[...])
    h = _swish(jnp.dot(h, w3_ref[...], preferred_element_type=jnp.float32)
               + b3_ref[...])
    o_ref[...] = jnp.dot(h, wo_ref[...], preferred_element_type=jnp.float32)


def _mlp(partial_list, W1, b1, W2, b2, W3, b3, W_out):
    O = W_out.shape[1]
    return pl.pallas_call(
        _mlp_body,
        grid=(_N // _NB,),
        in_specs=(
            [pl.BlockSpec((2, _NB, _D), lambda i: (0, i, 0))] * _NC
            + [
                pl.BlockSpec((_D, _D), lambda i: (0, 0)),
                pl.BlockSpec((1, _D), lambda i: (0, 0)),
                pl.BlockSpec((_D, _D), lambda i: (0, 0)),
                pl.BlockSpec((1, _D), lambda i: (0, 0)),
                pl.BlockSpec((_D, _D), lambda i: (0, 0)),
                pl.BlockSpec((1, _D), lambda i: (0, 0)),
                pl.BlockSpec((_D, O), lambda i: (0, 0)),
            ]
        ),
        out_specs=pl.BlockSpec((_NB, O), lambda i: (i, 0)),
        out_shape=jax.ShapeDtypeStruct((_N, O), jnp.float32),
    )(*partial_list, W1, b1.reshape(1, _D), W2, b2.reshape(1, _D), W3,
      b3.reshape(1, _D), W_out)


def kernel(x, rbf, idx_i, num_nodes, W_rbf, W1, b1, W2, b2, W3, b3, W_out):
    idx = idx_i.astype(jnp.int32).reshape(_NROWS, _B)
    rbf_t = rbf.T
    partial_list = []
    for k in range(_NC):
        h_k = _edge_embed(rbf_t, x, W_rbf, k)
        partial_list.append(
            _make_scatter_kernel(k * _CROWS)(h_k.reshape(_CROWS, _B, _D), idx))
    return _mlp(partial_list, W1, b1, W2, b2, W3, b3, W_out)
